# baseline (speedup 1.0000x reference)
import jax
import jax.numpy as jnp
from jax.experimental import pallas as pl
from jax.experimental.pallas import tpu as pltpu


def _copy_kernel(xd_ref, xp_ref, outd_ref, outp_ref):
    outd_ref[...] = jnp.minimum(xd_ref[...], 1.0)
    outp_ref[...] = jnp.minimum(xp_ref[...], 1.0)


@jax.jit
def kernel(diagnosis_x, procedure_x, lens, target_diagnoses, target_procedures,
           Wd1, bd1, Wd2, bd2, Wp1, bp1, Wp2, bp2):
    b, t, dnum = diagnosis_x.shape
    pnum = procedure_x.shape[-1]
    xd = diagnosis_x.reshape(b, t * dnum)
    xp = procedure_x.reshape(b, t * pnum)
    bb = 8
    grid = (b // bb,)
    big = lambda w: pl.BlockSpec((bb, w), lambda i: (i, 0))
    outd, outp = pl.pallas_call(
        _copy_kernel,
        grid=grid,
        in_specs=[big(t * dnum), big(t * pnum)],
        out_specs=[big(t * dnum), big(t * pnum)],
        out_shape=[
            jax.ShapeDtypeStruct((b, t * dnum), jnp.float32),
            jax.ShapeDtypeStruct((b, t * pnum), jnp.float32),
        ],
        compiler_params=pltpu.CompilerParams(
            dimension_semantics=("parallel",)),
    )(xd, xp)
    return (outd.reshape(b, t, dnum), outp.reshape(b, t, pnum))


# speedup vs baseline: 1.1057x; 1.1057x over previous
import functools

import jax
import jax.numpy as jnp
from jax.experimental import pallas as pl
from jax.experimental.pallas import tpu as pltpu

_BB = 2
_DEPTH = 8


def _in_copy(x_hbm, scratch, sem, chunk, slot, bb):
    return pltpu.make_async_copy(
        x_hbm.at[pl.ds(chunk * bb, bb)], scratch.at[slot], sem.at[slot])


def _out_copy(scratch, out_hbm, sem, chunk, slot, bb):
    return pltpu.make_async_copy(
        scratch.at[slot], out_hbm.at[pl.ds(chunk * bb, bb)], sem.at[slot])


def _fused_kernel(xd_hbm, xp_hbm, wd1_ref, wp1_ref, wd2_ref, wp2_ref,
                  bd1_ref, bp1_ref, bd2_ref, bp2_ref,
                  lens_ref, td_ref, tp_ref,
                  outd_hbm, outp_hbm,
                  xd_s, xp_s, od_s, op_s,
                  ind_sem, inp_sem, outd_sem, outp_sem,
                  *, bb, t, nsteps, depth):
    i = pl.program_id(0)
    slot = jax.lax.rem(i, depth)

    @pl.when(i == 0)
    def _():
        for k in range(depth - 1):
            _in_copy(xd_hbm, xd_s, ind_sem, k, k, bb).start()
            _in_copy(xp_hbm, xp_s, inp_sem, k, k, bb).start()

    j = i + depth - 1
    jslot = jax.lax.rem(j, depth)

    @pl.when(j < nsteps)
    def _():
        _in_copy(xd_hbm, xd_s, ind_sem, j, jslot, bb).start()
        _in_copy(xp_hbm, xp_s, inp_sem, j, jslot, bb).start()

    _in_copy(xd_hbm, xd_s, ind_sem, i, slot, bb).wait()
    _in_copy(xp_hbm, xp_s, inp_sem, i, slot, bb).wait()

    @pl.when(i >= depth)
    def _():
        _out_copy(od_s, outd_hbm, outd_sem, i - depth, slot, bb).wait()
        _out_copy(op_s, outp_hbm, outp_sem, i - depth, slot, bb).wait()

    lens_blk = lens_ref[pl.ds(i * bb, bb), 0]
    tmask = jax.lax.broadcasted_iota(jnp.int32, (bb, t), 1) < lens_blk[:, None]

    def branch(x_s, w1_ref, w2_ref, b1_ref, b2_ref, tgt_ref, out_s, width):
        x = x_s[slot]
        x2 = x.reshape(bb * t, width)
        h = jnp.tanh(
            jax.lax.dot_general(
                x2, w1_ref[...], (((1,), (0,)), ((), ())),
                preferred_element_type=jnp.float32) + b1_ref[...])
        s = jnp.sum(h * w2_ref[...], axis=1) + b2_ref[0, 0]
        s = s.reshape(bb, t)
        s = jnp.where(tmask, s, -1e9)
        m = jnp.max(s, axis=1, keepdims=True)
        e = jnp.exp(s - m)
        p = e / jnp.sum(e, axis=1, keepdims=True)
        tgt = tgt_ref[pl.ds(i * bb, bb), 0]
        onehot = (jax.lax.broadcasted_iota(jnp.int32, (bb, width), 1)
                  == tgt[:, None]).astype(jnp.float32)
        out_s[slot] = jnp.minimum(x + p[:, :, None] * onehot[:, None, :], 1.0)

    branch(xd_s, wd1_ref, wd2_ref, bd1_ref, bd2_ref, td_ref, od_s,
           xd_s.shape[-1])
    branch(xp_s, wp1_ref, wp2_ref, bp1_ref, bp2_ref, tp_ref, op_s,
           xp_s.shape[-1])

    _out_copy(od_s, outd_hbm, outd_sem, i, slot, bb).start()
    _out_copy(op_s, outp_hbm, outp_sem, i, slot, bb).start()

    @pl.when(i == nsteps - 1)
    def _():
        for s_ in range(depth):
            c = nsteps - depth + s_
            _out_copy(od_s, outd_hbm, outd_sem, c, c % depth, bb).wait()
            _out_copy(op_s, outp_hbm, outp_sem, c, c % depth, bb).wait()


@jax.jit
def kernel(diagnosis_x, procedure_x, lens, target_diagnoses, target_procedures,
           Wd1, bd1, Wd2, bd2, Wp1, bp1, Wp2, bp2):
    b, t, dnum = diagnosis_x.shape
    pnum = procedure_x.shape[-1]
    adim = Wd1.shape[-1]
    bb = _BB
    depth = _DEPTH
    nsteps = b // bb
    grid = (nsteps,)

    lens2 = lens.astype(jnp.int32).reshape(b, 1)
    td2 = target_diagnoses.astype(jnp.int32).reshape(b, 1)
    tp2 = target_procedures.astype(jnp.int32).reshape(b, 1)
    wd2r = Wd2.reshape(1, adim)
    wp2r = Wp2.reshape(1, adim)
    bd1r = bd1.reshape(1, adim)
    bp1r = bp1.reshape(1, adim)
    bd2r = bd2.reshape(1, 1)
    bp2r = bp2.reshape(1, 1)

    hbm = pl.BlockSpec(memory_space=pl.ANY)
    vfull = lambda shape: pl.BlockSpec(shape, lambda i: (0,) * len(shape))

    outd, outp = pl.pallas_call(
        functools.partial(_fused_kernel, bb=bb, t=t, nsteps=nsteps,
                          depth=depth),
        grid=grid,
        in_specs=[
            hbm, hbm,
            vfull((dnum, adim)), vfull((pnum, adim)),
            vfull((1, adim)), vfull((1, adim)),
            vfull((1, adim)), vfull((1, adim)),
            vfull((1, 1)), vfull((1, 1)),
            vfull((b, 1)), vfull((b, 1)), vfull((b, 1)),
        ],
        out_specs=[hbm, hbm],
        out_shape=[
            jax.ShapeDtypeStruct((b, t, dnum), jnp.float32),
            jax.ShapeDtypeStruct((b, t, pnum), jnp.float32),
        ],
        scratch_shapes=[
            pltpu.VMEM((depth, bb, t, dnum), jnp.float32),
            pltpu.VMEM((depth, bb, t, pnum), jnp.float32),
            pltpu.VMEM((depth, bb, t, dnum), jnp.float32),
            pltpu.VMEM((depth, bb, t, pnum), jnp.float32),
            pltpu.SemaphoreType.DMA((depth,)),
            pltpu.SemaphoreType.DMA((depth,)),
            pltpu.SemaphoreType.DMA((depth,)),
            pltpu.SemaphoreType.DMA((depth,)),
        ],
        compiler_params=pltpu.CompilerParams(
            dimension_semantics=("arbitrary",)),
    )(diagnosis_x, procedure_x, Wd1, Wp1, wd2r, wp2r,
      bd1r, bp1r, bd2r, bp2r, lens2, td2, tp2)
    return (outd, outp)
